# baseline (speedup 1.0000x reference)
import functools

import jax
import jax.numpy as jnp
from jax import lax
from jax.experimental import pallas as pl
from jax.experimental.pallas import tpu as pltpu
from jax.experimental.pallas import tpu_sc as plsc


def _xw_body(x_ref, w_ref, o_ref):
    o_ref[...] = jnp.dot(x_ref[...], w_ref[...],
                         preferred_element_type=jnp.float32)


def _stage_a(x, w_enc, bm):
    n, d = x.shape
    return pl.pallas_call(
        _xw_body,
        grid=(n // bm,),
        in_specs=[
            pl.BlockSpec((bm, d), lambda m: (m, 0)),
            pl.BlockSpec((d, d), lambda m: (0, 0)),
        ],
        out_specs=pl.BlockSpec((bm, d), lambda m: (m, 0)),
        out_shape=jax.ShapeDtypeStruct((n, d), jnp.float32),
    )(x, w_enc)


def _st_body(adj_ref, xw_ref, wm_ref, b_ref, o_ref):
    h = jnp.dot(adj_ref[...], xw_ref[...],
                preferred_element_type=jnp.float32)
    o_ref[...] = jnp.maximum(h, 0.0) @ wm_ref[...] + b_ref[...]


def _stage_b(adj, xw, wm2, bias2, bm):
    n, d = xw.shape
    return pl.pallas_call(
        _st_body,
        grid=(n // bm,),
        in_specs=[
            pl.BlockSpec((bm, n), lambda m: (m, 0)),
            pl.BlockSpec((n, d), lambda m: (0, 0)),
            pl.BlockSpec((d, 2), lambda m: (0, 0)),
            pl.BlockSpec((1, 2), lambda m: (0, 0)),
        ],
        out_specs=pl.BlockSpec((bm, 2), lambda m: (m, 0)),
        out_shape=jax.ShapeDtypeStruct((n, 2), jnp.float32),
        compiler_params=pltpu.CompilerParams(
            dimension_semantics=("arbitrary",),
        ),
    )(adj, xw, wm2, bias2)


def _make_sc_gather(n, e):
    info = plsc.get_sparse_core_info()
    nc, ns, nl = info.num_cores, info.num_subcores, info.num_lanes
    nw = nc * ns
    epw = e // nw
    mesh = plsc.VectorSubcoreMesh(core_axis_name="c", subcore_axis_name="s")

    @functools.partial(
        pl.kernel,
        out_type=jax.ShapeDtypeStruct((e,), jnp.float32),
        mesh=mesh,
        scratch_types=[
            pltpu.VMEM((n,), jnp.float32),
            pltpu.VMEM((n,), jnp.float32),
            pltpu.VMEM((epw,), jnp.int32),
            pltpu.VMEM((epw,), jnp.int32),
            pltpu.VMEM((epw,), jnp.float32),
        ],
        compiler_params=pltpu.CompilerParams(needs_layout_passes=False),
    )
    def sc_gather(s_hbm, t_hbm, src_hbm, dst_hbm, out_hbm,
                  s_v, t_v, si_v, di_v, o_v):
        wid = lax.axis_index("s") * nc + lax.axis_index("c")
        base = wid * epw
        pltpu.sync_copy(s_hbm, s_v)
        pltpu.sync_copy(t_hbm, t_v)
        pltpu.sync_copy(src_hbm.at[pl.ds(base, epw)], si_v)
        pltpu.sync_copy(dst_hbm.at[pl.ds(base, epw)], di_v)

        def body(i, carry):
            off = i * nl
            sv = plsc.load_gather(s_v, [si_v[pl.ds(off, nl)]])
            tv = plsc.load_gather(t_v, [di_v[pl.ds(off, nl)]])
            o_v[pl.ds(off, nl)] = sv + tv
            return carry

        lax.fori_loop(0, epw // nl, body, 0)
        pltpu.sync_copy(o_v, out_hbm.at[pl.ds(base, epw)])

    return sc_gather


def kernel(x, adj, edge_index, W_enc, W_mlp, b_mlp):
    n, d = x.shape
    e = edge_index.shape[1]

    wm2 = jnp.concatenate([W_mlp[:d], W_mlp[d:]], axis=1)
    bias2 = jnp.concatenate([b_mlp, jnp.zeros_like(b_mlp)])
    bias2 = bias2.reshape(1, 2)

    xw = _stage_a(x, W_enc, bm=2000)
    st = _stage_b(adj, xw, wm2, bias2, bm=200)

    s = st[:, 0]
    t = st[:, 1]
    src = edge_index[0]
    dst = edge_index[1]

    out = _make_sc_gather(n, e)(s, t, src, dst)
    return out.reshape(e, 1)


# speedup vs baseline: 1.0303x; 1.0303x over previous
import functools

import jax
import jax.numpy as jnp
from jax import lax
from jax.experimental import pallas as pl
from jax.experimental.pallas import tpu as pltpu
from jax.experimental.pallas import tpu_sc as plsc


def _xw_body(x_ref, w_ref, o_ref):
    o_ref[...] = jnp.dot(x_ref[...], w_ref[...],
                         preferred_element_type=jnp.float32)


def _stage_a(x, w_enc, bm):
    n, d = x.shape
    return pl.pallas_call(
        _xw_body,
        grid=(n // bm,),
        in_specs=[
            pl.BlockSpec((bm, d), lambda m: (m, 0)),
            pl.BlockSpec((d, d), lambda m: (0, 0)),
        ],
        out_specs=pl.BlockSpec((bm, d), lambda m: (m, 0)),
        out_shape=jax.ShapeDtypeStruct((n, d), jnp.float32),
    )(x, w_enc)


def _st_body(adj_t_ref, adj_b_ref, xw_ref, wm_ref, b_ref, ot_ref, ob_ref):
    ht = jnp.dot(adj_t_ref[...], xw_ref[...],
                 preferred_element_type=jnp.float32)
    hb = jnp.dot(adj_b_ref[...], xw_ref[...],
                 preferred_element_type=jnp.float32)
    ot_ref[...] = jnp.maximum(ht, 0.0) @ wm_ref[...] + b_ref[...]
    ob_ref[...] = jnp.maximum(hb, 0.0) @ wm_ref[...] + b_ref[...]


def _stage_b(adj, xw, wm2, bias2, bm):
    n, d = xw.shape
    nh = n // 2
    nsteps = nh // bm
    out_sd = jax.ShapeDtypeStruct((nh, 2), jnp.float32)
    return pl.pallas_call(
        _st_body,
        grid=(nsteps,),
        in_specs=[
            pl.BlockSpec((bm, n), lambda m: (m, 0)),
            pl.BlockSpec((bm, n), lambda m: (m + nsteps, 0)),
            pl.BlockSpec((n, d), lambda m: (0, 0)),
            pl.BlockSpec((d, 2), lambda m: (0, 0)),
            pl.BlockSpec((1, 2), lambda m: (0, 0)),
        ],
        out_specs=[
            pl.BlockSpec((bm, 2), lambda m: (m, 0)),
            pl.BlockSpec((bm, 2), lambda m: (m, 0)),
        ],
        out_shape=[out_sd, out_sd],
        compiler_params=pltpu.CompilerParams(
            dimension_semantics=("arbitrary",),
        ),
    )(adj, adj, xw, wm2, bias2)


def _make_sc_gather(n, e):
    info = plsc.get_sparse_core_info()
    nc, ns, nl = info.num_cores, info.num_subcores, info.num_lanes
    nw = nc * ns
    epw = e // nw
    mesh = plsc.VectorSubcoreMesh(core_axis_name="c", subcore_axis_name="s")

    @functools.partial(
        pl.kernel,
        out_type=jax.ShapeDtypeStruct((e,), jnp.float32),
        mesh=mesh,
        scratch_types=[
            pltpu.VMEM((n,), jnp.float32),
            pltpu.VMEM((n,), jnp.float32),
            pltpu.VMEM((epw,), jnp.int32),
            pltpu.VMEM((epw,), jnp.int32),
            pltpu.VMEM((epw,), jnp.float32),
        ],
        compiler_params=pltpu.CompilerParams(needs_layout_passes=False),
    )
    def sc_gather(s_hbm, t_hbm, src_hbm, dst_hbm, out_hbm,
                  s_v, t_v, si_v, di_v, o_v):
        wid = lax.axis_index("s") * nc + lax.axis_index("c")
        base = wid * epw
        pltpu.sync_copy(s_hbm, s_v)
        pltpu.sync_copy(t_hbm, t_v)
        pltpu.sync_copy(src_hbm.at[pl.ds(base, epw)], si_v)
        pltpu.sync_copy(dst_hbm.at[pl.ds(base, epw)], di_v)

        def body(i, carry):
            off = i * nl
            sv = plsc.load_gather(s_v, [si_v[pl.ds(off, nl)]])
            tv = plsc.load_gather(t_v, [di_v[pl.ds(off, nl)]])
            o_v[pl.ds(off, nl)] = sv + tv
            return carry

        lax.fori_loop(0, epw // nl, body, 0)
        pltpu.sync_copy(o_v, out_hbm.at[pl.ds(base, epw)])

    return sc_gather


def kernel(x, adj, edge_index, W_enc, W_mlp, b_mlp):
    n, d = x.shape
    e = edge_index.shape[1]

    wm2 = jnp.concatenate([W_mlp[:d], W_mlp[d:]], axis=1)
    bias2 = jnp.concatenate([b_mlp, jnp.zeros_like(b_mlp)])
    bias2 = bias2.reshape(1, 2)

    xw = _stage_a(x, W_enc, bm=2000)
    st_top, st_bot = _stage_b(adj, xw, wm2, bias2, bm=200)
    st = jnp.concatenate([st_top, st_bot], axis=0)

    s = st[:, 0]
    t = st[:, 1]
    src = edge_index[0]
    dst = edge_index[1]

    out = _make_sc_gather(n, e)(s, t, src, dst)
    return out.reshape(e, 1)


# speedup vs baseline: 1.0427x; 1.0121x over previous
import functools

import jax
import jax.numpy as jnp
from jax import lax
from jax.experimental import pallas as pl
from jax.experimental.pallas import tpu as pltpu
from jax.experimental.pallas import tpu_sc as plsc


def _xw_body(x_ref, w_ref, o_ref):
    o_ref[...] = jnp.dot(x_ref[...], w_ref[...],
                         preferred_element_type=jnp.float32)


def _stage_a(x, w_enc, bm):
    n, d = x.shape
    return pl.pallas_call(
        _xw_body,
        grid=(n // bm,),
        in_specs=[
            pl.BlockSpec((bm, d), lambda m: (m, 0)),
            pl.BlockSpec((d, d), lambda m: (0, 0)),
        ],
        out_specs=pl.BlockSpec((bm, d), lambda m: (m, 0)),
        out_shape=jax.ShapeDtypeStruct((n, d), jnp.float32),
    )(x, w_enc)


def _st_body(adj_t_ref, adj_b_ref, xw_ref, wm_ref, b_ref, ot_ref, ob_ref):
    ht = jnp.dot(adj_t_ref[...], xw_ref[...],
                 preferred_element_type=jnp.float32)
    hb = jnp.dot(adj_b_ref[...], xw_ref[...],
                 preferred_element_type=jnp.float32)
    ot_ref[...] = jnp.maximum(ht, 0.0) @ wm_ref[...] + b_ref[...]
    ob_ref[...] = jnp.maximum(hb, 0.0) @ wm_ref[...] + b_ref[...]


def _stage_b(adj, xw, wm2, bias2, bm):
    n, d = xw.shape
    nh = n // 2
    nsteps = nh // bm
    out_sd = jax.ShapeDtypeStruct((nh, 2), jnp.float32)
    return pl.pallas_call(
        _st_body,
        grid=(nsteps,),
        in_specs=[
            pl.BlockSpec((bm, n), lambda m: (m, 0)),
            pl.BlockSpec((bm, n), lambda m: (m + nsteps, 0)),
            pl.BlockSpec((n, d), lambda m: (0, 0)),
            pl.BlockSpec((d, 2), lambda m: (0, 0)),
            pl.BlockSpec((1, 2), lambda m: (0, 0)),
        ],
        out_specs=[
            pl.BlockSpec((bm, 2), lambda m: (m, 0)),
            pl.BlockSpec((bm, 2), lambda m: (m, 0)),
        ],
        out_shape=[out_sd, out_sd],
        compiler_params=pltpu.CompilerParams(
            dimension_semantics=("arbitrary",),
        ),
    )(adj, adj, xw, wm2, bias2)


def _make_sc_gather(n, e):
    info = plsc.get_sparse_core_info()
    nc, ns, nl = info.num_cores, info.num_subcores, info.num_lanes
    nw = nc * ns
    epw = e // nw
    mesh = plsc.VectorSubcoreMesh(core_axis_name="c", subcore_axis_name="s")

    @functools.partial(
        pl.kernel,
        out_type=jax.ShapeDtypeStruct((e,), jnp.float32),
        mesh=mesh,
        scratch_types=[
            pltpu.VMEM((n,), jnp.float32),
            pltpu.VMEM((n,), jnp.float32),
            pltpu.VMEM((epw,), jnp.int32),
            pltpu.VMEM((epw,), jnp.int32),
            pltpu.VMEM((epw,), jnp.float32),
        ],
        compiler_params=pltpu.CompilerParams(needs_layout_passes=False),
    )
    def sc_gather(s_hbm, t_hbm, src_hbm, dst_hbm, out_hbm,
                  s_v, t_v, si_v, di_v, o_v):
        wid = lax.axis_index("s") * nc + lax.axis_index("c")
        base = wid * epw
        pltpu.sync_copy(s_hbm, s_v)
        pltpu.sync_copy(t_hbm, t_v)
        pltpu.sync_copy(src_hbm.at[pl.ds(base, epw)], si_v)
        pltpu.sync_copy(dst_hbm.at[pl.ds(base, epw)], di_v)

        @plsc.parallel_loop(0, epw // nl, unroll=8)
        def body(i):
            off = i * nl
            sv = plsc.load_gather(s_v, [si_v[pl.ds(off, nl)]])
            tv = plsc.load_gather(t_v, [di_v[pl.ds(off, nl)]])
            o_v[pl.ds(off, nl)] = sv + tv
        pltpu.sync_copy(o_v, out_hbm.at[pl.ds(base, epw)])

    return sc_gather


def kernel(x, adj, edge_index, W_enc, W_mlp, b_mlp):
    n, d = x.shape
    e = edge_index.shape[1]

    wm2 = jnp.concatenate([W_mlp[:d], W_mlp[d:]], axis=1)
    bias2 = jnp.concatenate([b_mlp, jnp.zeros_like(b_mlp)])
    bias2 = bias2.reshape(1, 2)

    xw = _stage_a(x, W_enc, bm=2000)
    st_top, st_bot = _stage_b(adj, xw, wm2, bias2, bm=200)
    st = jnp.concatenate([st_top, st_bot], axis=0)

    s = st[:, 0]
    t = st[:, 1]
    src = edge_index[0]
    dst = edge_index[1]

    out = _make_sc_gather(n, e)(s, t, src, dst)
    return out.reshape(e, 1)


# speedup vs baseline: 1.0892x; 1.0446x over previous
import functools

import jax
import jax.numpy as jnp
from jax import lax
from jax.experimental import pallas as pl
from jax.experimental.pallas import tpu as pltpu
from jax.experimental.pallas import tpu_sc as plsc


def _st_body(x_ref, we_ref, adj_t_ref, adj_b_ref, wm_ref, b_ref,
             ot_ref, ob_ref, xw_s):
    @pl.when(pl.program_id(0) == 0)
    def _():
        xw_s[...] = jnp.dot(x_ref[...], we_ref[...],
                            preferred_element_type=jnp.float32)

    ht = jnp.dot(adj_t_ref[...], xw_s[...],
                 preferred_element_type=jnp.float32)
    hb = jnp.dot(adj_b_ref[...], xw_s[...],
                 preferred_element_type=jnp.float32)
    ot_ref[...] = jnp.maximum(ht, 0.0) @ wm_ref[...] + b_ref[...]
    ob_ref[...] = jnp.maximum(hb, 0.0) @ wm_ref[...] + b_ref[...]


def _stage_tc(x, adj, w_enc, wm2, bias2, bm):
    n, d = x.shape
    nh = n // 2
    nsteps = nh // bm
    out_sd = jax.ShapeDtypeStruct((nh, 2), jnp.float32)
    return pl.pallas_call(
        _st_body,
        grid=(nsteps,),
        in_specs=[
            pl.BlockSpec((n, d), lambda m: (0, 0)),
            pl.BlockSpec((d, d), lambda m: (0, 0)),
            pl.BlockSpec((bm, n), lambda m: (m, 0)),
            pl.BlockSpec((bm, n), lambda m: (m + nsteps, 0)),
            pl.BlockSpec((d, 2), lambda m: (0, 0)),
            pl.BlockSpec((1, 2), lambda m: (0, 0)),
        ],
        out_specs=[
            pl.BlockSpec((bm, 2), lambda m: (m, 0)),
            pl.BlockSpec((bm, 2), lambda m: (m, 0)),
        ],
        out_shape=[out_sd, out_sd],
        scratch_shapes=[pltpu.VMEM((n, d), jnp.float32)],
        compiler_params=pltpu.CompilerParams(
            dimension_semantics=("arbitrary",),
        ),
    )(x, w_enc, adj, adj, wm2, bias2)


def _make_sc_gather(n, e):
    info = plsc.get_sparse_core_info()
    nc, ns, nl = info.num_cores, info.num_subcores, info.num_lanes
    nw = nc * ns
    epw = e // nw
    mesh = plsc.VectorSubcoreMesh(core_axis_name="c", subcore_axis_name="s")

    @functools.partial(
        pl.kernel,
        out_type=jax.ShapeDtypeStruct((e,), jnp.float32),
        mesh=mesh,
        scratch_types=[
            pltpu.VMEM((2 * n,), jnp.float32),
            pltpu.VMEM((epw,), jnp.int32),
            pltpu.VMEM((epw,), jnp.int32),
            pltpu.VMEM((epw,), jnp.float32),
        ],
        compiler_params=pltpu.CompilerParams(needs_layout_passes=False),
    )
    def sc_gather(st_hbm, eif_hbm, out_hbm, st_v, si_v, di_v, o_v):
        wid = lax.axis_index("s") * nc + lax.axis_index("c")
        base = wid * epw
        pltpu.sync_copy(st_hbm, st_v)
        pltpu.sync_copy(eif_hbm.at[pl.ds(base, epw)], si_v)
        pltpu.sync_copy(eif_hbm.at[pl.ds(e + base, epw)], di_v)

        one = jnp.ones((nl,), jnp.int32)

        @plsc.parallel_loop(0, epw // nl, unroll=8)
        def body(i):
            off = i * nl
            sidx = si_v[pl.ds(off, nl)] * 2
            didx = di_v[pl.ds(off, nl)] * 2 + one
            sv = plsc.load_gather(st_v, [sidx])
            tv = plsc.load_gather(st_v, [didx])
            o_v[pl.ds(off, nl)] = sv + tv

        pltpu.sync_copy(o_v, out_hbm.at[pl.ds(base, epw)])

    return sc_gather


def kernel(x, adj, edge_index, W_enc, W_mlp, b_mlp):
    n, d = x.shape
    e = edge_index.shape[1]

    wm2 = jnp.concatenate([W_mlp[:d], W_mlp[d:]], axis=1)
    bias2 = jnp.concatenate([b_mlp, jnp.zeros_like(b_mlp)])
    bias2 = bias2.reshape(1, 2)

    st_top, st_bot = _stage_tc(x, adj, W_enc, wm2, bias2, bm=200)
    st_flat = jnp.concatenate([st_top, st_bot], axis=0).reshape(-1)
    ei_flat = edge_index.reshape(-1)

    out = _make_sc_gather(n, e)(st_flat, ei_flat)
    return out.reshape(e, 1)


# speedup vs baseline: 1.1003x; 1.0102x over previous
import functools

import jax
import jax.numpy as jnp
from jax import lax
from jax.experimental import pallas as pl
from jax.experimental.pallas import tpu as pltpu
from jax.experimental.pallas import tpu_sc as plsc


def _st_body(x_ref, we_ref, adj_t_ref, adj_b_ref, wm_ref, b_ref,
             ot_ref, ob_ref, xw_s):
    @pl.when(pl.program_id(0) == 0)
    def _():
        xw_s[...] = jnp.dot(x_ref[...], we_ref[...],
                            preferred_element_type=jnp.float32)

    ht = jnp.dot(adj_t_ref[...], xw_s[...],
                 preferred_element_type=jnp.float32)
    hb = jnp.dot(adj_b_ref[...], xw_s[...],
                 preferred_element_type=jnp.float32)
    ot_ref[...] = jnp.maximum(ht, 0.0) @ wm_ref[...] + b_ref[...]
    ob_ref[...] = jnp.maximum(hb, 0.0) @ wm_ref[...] + b_ref[...]


def _stage_tc(x, adj, w_enc, wm2, bias2, bm):
    n, d = x.shape
    nh = n // 2
    nsteps = nh // bm
    out_sd = jax.ShapeDtypeStruct((nh, 2), jnp.float32)
    return pl.pallas_call(
        _st_body,
        grid=(nsteps,),
        in_specs=[
            pl.BlockSpec((n, d), lambda m: (0, 0)),
            pl.BlockSpec((d, d), lambda m: (0, 0)),
            pl.BlockSpec((bm, n), lambda m: (m, 0)),
            pl.BlockSpec((bm, n), lambda m: (m + nsteps, 0)),
            pl.BlockSpec((d, 2), lambda m: (0, 0)),
            pl.BlockSpec((1, 2), lambda m: (0, 0)),
        ],
        out_specs=[
            pl.BlockSpec((bm, 2), lambda m: (m, 0)),
            pl.BlockSpec((bm, 2), lambda m: (m, 0)),
        ],
        out_shape=[out_sd, out_sd],
        scratch_shapes=[pltpu.VMEM((n, d), jnp.float32)],
        compiler_params=pltpu.CompilerParams(
            dimension_semantics=("arbitrary",),
        ),
    )(x, w_enc, adj, adj, wm2, bias2)


def _make_sc_gather(n, e):
    info = plsc.get_sparse_core_info()
    nc, ns, nl = info.num_cores, info.num_subcores, info.num_lanes
    nw = nc * ns
    epw = e // nw
    mesh = plsc.VectorSubcoreMesh(core_axis_name="c", subcore_axis_name="s")

    @functools.partial(
        pl.kernel,
        out_type=jax.ShapeDtypeStruct((e,), jnp.float32),
        mesh=mesh,
        scratch_types=[
            pltpu.VMEM((2 * n,), jnp.float32),
            pltpu.VMEM((epw,), jnp.int32),
            pltpu.VMEM((epw,), jnp.int32),
            pltpu.VMEM((epw,), jnp.float32),
            pltpu.SemaphoreType.DMA,
        ],
        compiler_params=pltpu.CompilerParams(needs_layout_passes=False),
    )
    def sc_gather(st_hbm, eif_hbm, out_hbm, st_v, si_v, di_v, o_v, sem):
        wid = lax.axis_index("s") * nc + lax.axis_index("c")
        base = wid * epw
        c1 = pltpu.async_copy(st_hbm, st_v, sem)
        c2 = pltpu.async_copy(eif_hbm.at[pl.ds(base, epw)], si_v, sem)
        c3 = pltpu.async_copy(eif_hbm.at[pl.ds(e + base, epw)], di_v, sem)
        c1.wait()
        c2.wait()
        c3.wait()

        one = jnp.ones((nl,), jnp.int32)

        @plsc.parallel_loop(0, epw // nl, unroll=16)
        def body(i):
            off = i * nl
            sidx = si_v[pl.ds(off, nl)] * 2
            didx = di_v[pl.ds(off, nl)] * 2 + one
            sv = plsc.load_gather(st_v, [sidx])
            tv = plsc.load_gather(st_v, [didx])
            o_v[pl.ds(off, nl)] = sv + tv

        pltpu.sync_copy(o_v, out_hbm.at[pl.ds(base, epw)])

    return sc_gather


def kernel(x, adj, edge_index, W_enc, W_mlp, b_mlp):
    n, d = x.shape
    e = edge_index.shape[1]

    wm2 = jnp.concatenate([W_mlp[:d], W_mlp[d:]], axis=1)
    bias2 = jnp.concatenate([b_mlp, jnp.zeros_like(b_mlp)])
    bias2 = bias2.reshape(1, 2)

    st_top, st_bot = _stage_tc(x, adj, W_enc, wm2, bias2, bm=200)
    st_flat = jnp.concatenate([st_top, st_bot], axis=0).reshape(-1)
    ei_flat = edge_index.reshape(-1)

    out = _make_sc_gather(n, e)(st_flat, ei_flat)
    return out.reshape(e, 1)
